# Initial kernel scaffold; baseline (speedup 1.0000x reference)
import jax
import jax.numpy as jnp
from jax.experimental import pallas as pl


def kernel(h, edge_index, e, W_embed, b_embed, Ws, bs, gammas, betas):
    raise NotImplementedError("write your pallas kernel here")


# speedup vs baseline: 4.1990x; 4.1990x over previous
import functools

import jax
import jax.numpy as jnp
from jax import lax
from jax.experimental import pallas as pl
from jax.experimental.pallas import tpu as pltpu
from jax.experimental.pallas import tpu_sc as plsc

N = 10000
E = 320000
D = 128
H = 64
NP = 10240
L = 4

NCORE = 2
NSUB = 16
EPT = E // NSUB
K = 80
NIT = EPT // K
RPT = NP // NSUB

_MESH = plsc.VectorSubcoreMesh(core_axis_name="c", subcore_axis_name="s")


DW = 128


def _deg_body(src_hbm, dst_hbm, ones_s_hbm, ones_d_hbm, z_hbm, deg_hbm,
              idx_s, idx_d, ones_s, ones_d, deg_sh):
    c = lax.axis_index("c")
    s = lax.axis_index("s")
    pltpu.sync_copy(ones_s_hbm, ones_s)
    pltpu.sync_copy(ones_d_hbm, ones_d)
    pltpu.sync_copy(z_hbm.at[pl.ds(s * RPT, RPT)],
                    deg_sh.at[pl.ds(s * RPT, RPT)])
    plsc.subcore_barrier()
    base = c * (E // NCORE) + s * (E // NCORE // NSUB)

    @pl.loop(0, E // NCORE // NSUB // K)
    def _(i):
        off = base + i * K
        pltpu.sync_copy(src_hbm.at[pl.ds(off, K)], idx_s)
        pltpu.sync_copy(dst_hbm.at[pl.ds(off, K)], idx_d)
        pltpu.sync_copy(ones_s, deg_sh.at[idx_s], add=True)
        pltpu.sync_copy(ones_d, deg_sh.at[idx_d], add=True)

    plsc.subcore_barrier()
    pltpu.sync_copy(deg_sh.at[pl.ds(s * RPT, RPT)],
                    deg_hbm.at[c, pl.ds(s * RPT, RPT)])


_deg_call = pl.kernel(
    _deg_body,
    out_type=jax.ShapeDtypeStruct((NCORE, NP, DW), jnp.float32),
    mesh=_MESH,
    scratch_types=[
        pltpu.VMEM((K,), jnp.int32),
        pltpu.VMEM((K,), jnp.int32),
        pltpu.VMEM((K, DW), jnp.float32),
        pltpu.VMEM((K, DW), jnp.float32),
        pltpu.VMEM_SHARED((NP, DW), jnp.float32),
    ],
)


EPC = E // NCORE
EPT2 = EPC // NSUB
NIT2 = EPT2 // K


def _agg_body(m_hbm, src_hbm, dst_hbm, z_hbm, agg_hbm, idx_s, idx_d, rows,
              agg_sh, sem):
    c = lax.axis_index("c")
    s = lax.axis_index("s")
    pltpu.sync_copy(z_hbm.at[pl.ds(s * RPT, RPT)],
                    agg_sh.at[pl.ds(s * RPT, RPT)])
    plsc.subcore_barrier()
    base = c * EPC + s * EPT2

    @pl.loop(0, NIT2)
    def _(i):
        off = base + i * K
        pltpu.sync_copy(src_hbm.at[pl.ds(off, K)], idx_s)
        pltpu.sync_copy(dst_hbm.at[pl.ds(off, K)], idx_d)
        pltpu.async_copy(m_hbm.at[idx_s], rows, sem).wait()
        pltpu.sync_copy(rows, agg_sh.at[idx_d], add=True)

    plsc.subcore_barrier()
    pltpu.sync_copy(agg_sh.at[pl.ds(s * RPT, RPT)],
                    agg_hbm.at[c, pl.ds(s * RPT, RPT)])


_agg_call = pl.kernel(
    _agg_body,
    out_type=jax.ShapeDtypeStruct((NCORE, NP, D), jnp.float32),
    mesh=_MESH,
    scratch_types=[
        pltpu.VMEM((K,), jnp.int32),
        pltpu.VMEM((K,), jnp.int32),
        pltpu.VMEM((K, D), jnp.float32),
        pltpu.VMEM_SHARED((NP, D), jnp.float32),
        pltpu.SemaphoreType.DMA,
    ],
)


R = 2000
G = N // R


def _norm(d):
    return jnp.where(d > 0, lax.rsqrt(jnp.maximum(d, 1.0)), 0.0)


def _prologue_body(h_ref, w_ref, b_ref, deg_ref, h0_ref, m_ref):
    h0 = jnp.dot(h_ref[...], w_ref[...],
                 preferred_element_type=jnp.float32) + b_ref[...]
    h0_ref[...] = h0
    ns = _norm(deg_ref[0, :, 0] + deg_ref[1, :, 0])
    m_ref[...] = h0 * ns[:, None]


_prologue_call = pl.pallas_call(
    _prologue_body,
    grid=(G,),
    in_specs=[
        pl.BlockSpec((R, D), lambda i: (i, 0)),
        pl.BlockSpec((D, D), lambda i: (0, 0)),
        pl.BlockSpec((1, D), lambda i: (0, 0)),
        pl.BlockSpec((NCORE, R, DW), lambda i: (0, i, 0)),
    ],
    out_specs=[
        pl.BlockSpec((R, D), lambda i: (i, 0)),
        pl.BlockSpec((R, D), lambda i: (i, 0)),
    ],
    out_shape=[
        jax.ShapeDtypeStruct((N, D), jnp.float32),
        jax.ShapeDtypeStruct((N, D), jnp.float32),
    ],
)


def _layer_body(is_last, agg_ref, h_ref, w_ref, b_ref, g_ref, bt_ref,
                deg_ref, *outs):
    nd = _norm(deg_ref[0, :, 1] + deg_ref[1, :, 1])
    a = (agg_ref[0] + agg_ref[1]) * nd[:, None]
    hl = (jnp.dot(a, w_ref[...], preferred_element_type=jnp.float32)
          + b_ref[...])
    hl = jnp.maximum(g_ref[...] * hl + bt_ref[...], 0.0)
    hout = h_ref[...] + hl
    if is_last:
        hg_ref, = outs

        @pl.when(pl.program_id(0) == 0)
        def _():
            hg_ref[...] = jnp.zeros_like(hg_ref)

        hg_ref[...] += jnp.sum(hout, axis=0, keepdims=True) * (1.0 / N)
    else:
        hout_ref, m_ref = outs
        hout_ref[...] = hout
        ns = _norm(deg_ref[0, :, 0] + deg_ref[1, :, 0])
        m_ref[...] = hout * ns[:, None]


def _make_layer(is_last):
    if is_last:
        out_shape = [jax.ShapeDtypeStruct((1, D), jnp.float32)]
        out_specs = [pl.BlockSpec((1, D), lambda i: (0, 0))]
    else:
        out_shape = [
            jax.ShapeDtypeStruct((N, D), jnp.float32),
            jax.ShapeDtypeStruct((N, D), jnp.float32),
        ]
        out_specs = [
            pl.BlockSpec((R, D), lambda i: (i, 0)),
            pl.BlockSpec((R, D), lambda i: (i, 0)),
        ]
    return pl.pallas_call(
        functools.partial(_layer_body, is_last),
        grid=(G,),
        in_specs=[
            pl.BlockSpec((NCORE, R, D), lambda i: (0, i, 0)),
            pl.BlockSpec((R, D), lambda i: (i, 0)),
            pl.BlockSpec((D, D), lambda i: (0, 0)),
            pl.BlockSpec((1, D), lambda i: (0, 0)),
            pl.BlockSpec((1, D), lambda i: (0, 0)),
            pl.BlockSpec((1, D), lambda i: (0, 0)),
            pl.BlockSpec((NCORE, R, DW), lambda i: (0, i, 0)),
        ],
        out_specs=out_specs,
        out_shape=out_shape,
    )


_layer_mid = _make_layer(False)
_layer_last = _make_layer(True)


def kernel(h, edge_index, e, W_embed, b_embed, Ws, bs, gammas, betas):
    del e
    ei = edge_index.astype(jnp.int32)
    src = ei[0]
    dst = ei[1]
    ones_s = jnp.zeros((K, DW), jnp.float32).at[:, 0].set(1.0)
    ones_d = jnp.zeros((K, DW), jnp.float32).at[:, 1].set(1.0)
    zeros = jnp.zeros((NP, D), jnp.float32)
    deg = _deg_call(src, dst, ones_s, ones_d, zeros)
    hcur, m = _prologue_call(h, W_embed, b_embed.reshape(1, D), deg)
    for l in range(L):
        agg = _agg_call(m, src, dst, zeros)
        b2 = bs[l].reshape(1, D)
        g2 = gammas[l].reshape(1, D)
        bt2 = betas[l].reshape(1, D)
        if l < L - 1:
            hcur, m = _layer_mid(agg, hcur, Ws[l], b2, g2, bt2, deg)
        else:
            hg, = _layer_last(agg, hcur, Ws[l], b2, g2, bt2, deg)
    return hg


# speedup vs baseline: 4.4259x; 1.0540x over previous
import functools

import jax
import jax.numpy as jnp
from jax import lax
from jax.experimental import pallas as pl
from jax.experimental.pallas import tpu as pltpu
from jax.experimental.pallas import tpu_sc as plsc

N = 10000
E = 320000
D = 128
H = 64
NP = 10240
L = 4

NCORE = 2
NSUB = 16
EPT = E // NSUB
K = 80
NIT = EPT // K
RPT = NP // NSUB

_MESH = plsc.VectorSubcoreMesh(core_axis_name="c", subcore_axis_name="s")


def _deg_body(src_hbm, dst_hbm, ones_hbm, z_hbm, deg_s_hbm, deg_d_hbm,
              idx_s, idx_d, ones_v, deg_s_sh, deg_d_sh):
    c = lax.axis_index("c")
    s = lax.axis_index("s")
    pltpu.sync_copy(ones_hbm, ones_v)
    pltpu.sync_copy(z_hbm.at[pl.ds(s * RPT, RPT)],
                    deg_s_sh.at[pl.ds(s * RPT, RPT)])
    pltpu.sync_copy(z_hbm.at[pl.ds(s * RPT, RPT)],
                    deg_d_sh.at[pl.ds(s * RPT, RPT)])
    plsc.subcore_barrier()
    base = c * EPC + s * EPT2

    @pl.loop(0, NIT2)
    def _(i):
        off = base + i * K
        pltpu.sync_copy(src_hbm.at[pl.ds(off, K)], idx_s)
        pltpu.sync_copy(dst_hbm.at[pl.ds(off, K)], idx_d)
        pltpu.sync_copy(ones_v, deg_s_sh.at[idx_s], add=True)
        pltpu.sync_copy(ones_v, deg_d_sh.at[idx_d], add=True)

    plsc.subcore_barrier()
    pltpu.sync_copy(deg_s_sh.at[pl.ds(s * RPT, RPT)],
                    deg_s_hbm.at[pl.ds(c * NP + s * RPT, RPT)])
    pltpu.sync_copy(deg_d_sh.at[pl.ds(s * RPT, RPT)],
                    deg_d_hbm.at[pl.ds(c * NP + s * RPT, RPT)])


_deg_call = pl.kernel(
    _deg_body,
    out_type=(jax.ShapeDtypeStruct((NCORE * NP,), jnp.float32),
              jax.ShapeDtypeStruct((NCORE * NP,), jnp.float32)),
    mesh=_MESH,
    scratch_types=[
        pltpu.VMEM((K,), jnp.int32),
        pltpu.VMEM((K,), jnp.int32),
        pltpu.VMEM((K,), jnp.float32),
        pltpu.VMEM_SHARED((NP,), jnp.float32),
        pltpu.VMEM_SHARED((NP,), jnp.float32),
    ],
)


EPC = E // NCORE
EPT2 = EPC // NSUB
NIT2 = EPT2 // K


def _agg_body(m_hbm, src_hbm, dst_hbm, z_hbm, agg_hbm, idx_s, idx_d, rows,
              agg_sh, sem):
    c = lax.axis_index("c")
    s = lax.axis_index("s")
    pltpu.sync_copy(z_hbm.at[pl.ds(s * RPT, RPT)],
                    agg_sh.at[pl.ds(s * RPT, RPT)])
    plsc.subcore_barrier()
    base = c * EPC + s * EPT2

    @pl.loop(0, NIT2)
    def _(i):
        off = base + i * K
        pltpu.sync_copy(src_hbm.at[pl.ds(off, K)], idx_s)
        pltpu.sync_copy(dst_hbm.at[pl.ds(off, K)], idx_d)
        pltpu.async_copy(m_hbm.at[idx_s], rows, sem).wait()
        pltpu.sync_copy(rows, agg_sh.at[idx_d], add=True)

    plsc.subcore_barrier()
    pltpu.sync_copy(agg_sh.at[pl.ds(s * RPT, RPT)],
                    agg_hbm.at[c, pl.ds(s * RPT, RPT)])


_agg_call = pl.kernel(
    _agg_body,
    out_type=jax.ShapeDtypeStruct((NCORE, NP, D), jnp.float32),
    mesh=_MESH,
    scratch_types=[
        pltpu.VMEM((K,), jnp.int32),
        pltpu.VMEM((K,), jnp.int32),
        pltpu.VMEM((K, D), jnp.float32),
        pltpu.VMEM_SHARED((NP, D), jnp.float32),
        pltpu.SemaphoreType.DMA,
    ],
)


R = 2000
G = N // R


def _norm(d):
    return jnp.where(d > 0, lax.rsqrt(jnp.maximum(d, 1.0)), 0.0)


def _prologue_body(h_ref, w_ref, b_ref, dsrc_ref, h0_ref, m_ref):
    h0 = jnp.dot(h_ref[...], w_ref[...],
                 preferred_element_type=jnp.float32) + b_ref[...]
    h0_ref[...] = h0
    ns = _norm(dsrc_ref[0, :, 0] + dsrc_ref[1, :, 0])
    m_ref[...] = h0 * ns[:, None]


_prologue_call = pl.pallas_call(
    _prologue_body,
    grid=(G,),
    in_specs=[
        pl.BlockSpec((R, D), lambda i: (i, 0)),
        pl.BlockSpec((D, D), lambda i: (0, 0)),
        pl.BlockSpec((1, D), lambda i: (0, 0)),
        pl.BlockSpec((NCORE, R, 1), lambda i: (0, i, 0)),
    ],
    out_specs=[
        pl.BlockSpec((R, D), lambda i: (i, 0)),
        pl.BlockSpec((R, D), lambda i: (i, 0)),
    ],
    out_shape=[
        jax.ShapeDtypeStruct((N, D), jnp.float32),
        jax.ShapeDtypeStruct((N, D), jnp.float32),
    ],
)


def _layer_body(is_last, agg_ref, h_ref, w_ref, b_ref, g_ref, bt_ref,
                dsrc_ref, ddst_ref, *outs):
    nd = _norm(ddst_ref[0, :, 0] + ddst_ref[1, :, 0])
    a = (agg_ref[0] + agg_ref[1]) * nd[:, None]
    hl = (jnp.dot(a, w_ref[...], preferred_element_type=jnp.float32)
          + b_ref[...])
    hl = jnp.maximum(g_ref[...] * hl + bt_ref[...], 0.0)
    hout = h_ref[...] + hl
    if is_last:
        hg_ref, = outs

        @pl.when(pl.program_id(0) == 0)
        def _():
            hg_ref[...] = jnp.zeros_like(hg_ref)

        hg_ref[...] += jnp.sum(hout, axis=0, keepdims=True) * (1.0 / N)
    else:
        hout_ref, m_ref = outs
        hout_ref[...] = hout
        ns = _norm(dsrc_ref[0, :, 0] + dsrc_ref[1, :, 0])
        m_ref[...] = hout * ns[:, None]


def _make_layer(is_last):
    if is_last:
        out_shape = [jax.ShapeDtypeStruct((1, D), jnp.float32)]
        out_specs = [pl.BlockSpec((1, D), lambda i: (0, 0))]
    else:
        out_shape = [
            jax.ShapeDtypeStruct((N, D), jnp.float32),
            jax.ShapeDtypeStruct((N, D), jnp.float32),
        ]
        out_specs = [
            pl.BlockSpec((R, D), lambda i: (i, 0)),
            pl.BlockSpec((R, D), lambda i: (i, 0)),
        ]
    return pl.pallas_call(
        functools.partial(_layer_body, is_last),
        grid=(G,),
        in_specs=[
            pl.BlockSpec((NCORE, R, D), lambda i: (0, i, 0)),
            pl.BlockSpec((R, D), lambda i: (i, 0)),
            pl.BlockSpec((D, D), lambda i: (0, 0)),
            pl.BlockSpec((1, D), lambda i: (0, 0)),
            pl.BlockSpec((1, D), lambda i: (0, 0)),
            pl.BlockSpec((1, D), lambda i: (0, 0)),
            pl.BlockSpec((NCORE, R, 1), lambda i: (0, i, 0)),
            pl.BlockSpec((NCORE, R, 1), lambda i: (0, i, 0)),
        ],
        out_specs=out_specs,
        out_shape=out_shape,
    )


_layer_mid = _make_layer(False)
_layer_last = _make_layer(True)


def kernel(h, edge_index, e, W_embed, b_embed, Ws, bs, gammas, betas):
    del e
    ei = edge_index.astype(jnp.int32)
    src = ei[0]
    dst = ei[1]
    ones_v = jnp.ones((K,), jnp.float32)
    zeros = jnp.zeros((NP, D), jnp.float32)
    z1 = jnp.zeros((NP,), jnp.float32)
    deg_s, deg_d = _deg_call(src, dst, ones_v, z1)
    deg_s = deg_s.reshape(NCORE, NP, 1)
    deg_d = deg_d.reshape(NCORE, NP, 1)
    hcur, m = _prologue_call(h, W_embed, b_embed.reshape(1, D), deg_s)
    for l in range(L):
        agg = _agg_call(m, src, dst, zeros)
        b2 = bs[l].reshape(1, D)
        g2 = gammas[l].reshape(1, D)
        bt2 = betas[l].reshape(1, D)
        if l < L - 1:
            hcur, m = _layer_mid(agg, hcur, Ws[l], b2, g2, bt2, deg_s,
                                 deg_d)
        else:
            hg, = _layer_last(agg, hcur, Ws[l], b2, g2, bt2, deg_s, deg_d)
    return hg


# speedup vs baseline: 9.1046x; 2.0571x over previous
import functools

import jax
import jax.numpy as jnp
from jax import lax
from jax.experimental import pallas as pl
from jax.experimental.pallas import tpu as pltpu
from jax.experimental.pallas import tpu_sc as plsc

N = 10000
E = 320000
D = 128
H = 64
NP = 10240
L = 4

NCORE = 2
NSUB = 16
EPT = E // NSUB
K = 80
NIT = EPT // K
RPT = NP // NSUB

_MESH = plsc.VectorSubcoreMesh(core_axis_name="c", subcore_axis_name="s")


def _deg_body(src_hbm, dst_hbm, ones_hbm, z_hbm, deg_s_hbm, deg_d_hbm,
              idx_s, idx_d, ones_v, deg_s_sh, deg_d_sh):
    c = lax.axis_index("c")
    s = lax.axis_index("s")
    pltpu.sync_copy(ones_hbm, ones_v)
    pltpu.sync_copy(z_hbm.at[pl.ds(s * RPT, RPT)],
                    deg_s_sh.at[pl.ds(s * RPT, RPT)])
    pltpu.sync_copy(z_hbm.at[pl.ds(s * RPT, RPT)],
                    deg_d_sh.at[pl.ds(s * RPT, RPT)])
    plsc.subcore_barrier()
    base = c * EPC + s * EPT2

    @pl.loop(0, NIT2)
    def _(i):
        off = base + i * K
        pltpu.sync_copy(src_hbm.at[pl.ds(off, K)], idx_s)
        pltpu.sync_copy(dst_hbm.at[pl.ds(off, K)], idx_d)
        pltpu.sync_copy(ones_v, deg_s_sh.at[idx_s], add=True)
        pltpu.sync_copy(ones_v, deg_d_sh.at[idx_d], add=True)

    plsc.subcore_barrier()
    pltpu.sync_copy(deg_s_sh.at[pl.ds(s * RPT, RPT)],
                    deg_s_hbm.at[pl.ds(c * NP + s * RPT, RPT)])
    pltpu.sync_copy(deg_d_sh.at[pl.ds(s * RPT, RPT)],
                    deg_d_hbm.at[pl.ds(c * NP + s * RPT, RPT)])


_deg_call = pl.kernel(
    _deg_body,
    out_type=(jax.ShapeDtypeStruct((NCORE * NP,), jnp.float32),
              jax.ShapeDtypeStruct((NCORE * NP,), jnp.float32)),
    mesh=_MESH,
    scratch_types=[
        pltpu.VMEM((K,), jnp.int32),
        pltpu.VMEM((K,), jnp.int32),
        pltpu.VMEM((K,), jnp.float32),
        pltpu.VMEM_SHARED((NP,), jnp.float32),
        pltpu.VMEM_SHARED((NP,), jnp.float32),
    ],
)


EPC = E // NCORE
EPT2 = EPC // NSUB
NIT2 = EPT2 // K


def _agg_body(m_hbm, src_hbm, dst_hbm, z_hbm, agg_hbm,
              sidx, d0, d1, r0, r1, agg_sh, sg0, sg1, si0, si1):
    c = lax.axis_index("c")
    s = lax.axis_index("s")
    pltpu.sync_copy(z_hbm.at[pl.ds(s * RPT, RPT)],
                    agg_sh.at[pl.ds(s * RPT, RPT)])
    base = c * EPC + s * EPT2
    pltpu.sync_copy(src_hbm.at[pl.ds(base, EPT2)], sidx)
    plsc.subcore_barrier()

    pltpu.async_copy(dst_hbm.at[pl.ds(base, K)], d0, si0)
    pltpu.async_copy(m_hbm.at[sidx.at[pl.ds(0, K)]], r0, sg0)

    @pl.loop(0, (NIT2 - 1) // 2)
    def _(j):
        i0 = 2 * j
        pltpu.async_copy(dst_hbm.at[pl.ds(base + (i0 + 1) * K, K)], d1, si1)
        pltpu.async_copy(m_hbm.at[sidx.at[pl.ds((i0 + 1) * K, K)]], r1, sg1)
        pltpu.make_async_copy(m_hbm.at[pl.ds(0, K)], r0, sg0).wait()
        pltpu.make_async_copy(dst_hbm.at[pl.ds(base, K)], d0, si0).wait()
        pltpu.sync_copy(r0, agg_sh.at[d0], add=True)
        pltpu.async_copy(dst_hbm.at[pl.ds(base + (i0 + 2) * K, K)], d0, si0)
        pltpu.async_copy(m_hbm.at[sidx.at[pl.ds((i0 + 2) * K, K)]], r0, sg0)
        pltpu.make_async_copy(m_hbm.at[pl.ds(0, K)], r1, sg1).wait()
        pltpu.make_async_copy(dst_hbm.at[pl.ds(base, K)], d1, si1).wait()
        pltpu.sync_copy(r1, agg_sh.at[d1], add=True)

    pltpu.make_async_copy(m_hbm.at[pl.ds(0, K)], r0, sg0).wait()
    pltpu.make_async_copy(dst_hbm.at[pl.ds(base, K)], d0, si0).wait()
    pltpu.sync_copy(r0, agg_sh.at[d0], add=True)

    plsc.subcore_barrier()
    pltpu.sync_copy(agg_sh.at[pl.ds(s * RPT, RPT)],
                    agg_hbm.at[c, pl.ds(s * RPT, RPT)])


_agg_call = pl.kernel(
    _agg_body,
    out_type=jax.ShapeDtypeStruct((NCORE, NP, D), jnp.float32),
    mesh=_MESH,
    scratch_types=[
        pltpu.VMEM((EPT2,), jnp.int32),
        pltpu.VMEM((K,), jnp.int32),
        pltpu.VMEM((K,), jnp.int32),
        pltpu.VMEM((K, D), jnp.float32),
        pltpu.VMEM((K, D), jnp.float32),
        pltpu.VMEM_SHARED((NP, D), jnp.float32),
        pltpu.SemaphoreType.DMA,
        pltpu.SemaphoreType.DMA,
        pltpu.SemaphoreType.DMA,
        pltpu.SemaphoreType.DMA,
    ],
)


R = 2000
G = N // R


def _norm(d):
    return jnp.where(d > 0, lax.rsqrt(jnp.maximum(d, 1.0)), 0.0)


def _prologue_body(h_ref, w_ref, b_ref, dsrc_ref, h0_ref, m_ref):
    h0 = jnp.dot(h_ref[...], w_ref[...],
                 preferred_element_type=jnp.float32) + b_ref[...]
    h0_ref[...] = h0
    ns = _norm(dsrc_ref[0, :, 0] + dsrc_ref[1, :, 0])
    m_ref[...] = h0 * ns[:, None]


_prologue_call = pl.pallas_call(
    _prologue_body,
    grid=(G,),
    in_specs=[
        pl.BlockSpec((R, D), lambda i: (i, 0)),
        pl.BlockSpec((D, D), lambda i: (0, 0)),
        pl.BlockSpec((1, D), lambda i: (0, 0)),
        pl.BlockSpec((NCORE, R, 1), lambda i: (0, i, 0)),
    ],
    out_specs=[
        pl.BlockSpec((R, D), lambda i: (i, 0)),
        pl.BlockSpec((R, D), lambda i: (i, 0)),
    ],
    out_shape=[
        jax.ShapeDtypeStruct((N, D), jnp.float32),
        jax.ShapeDtypeStruct((N, D), jnp.float32),
    ],
)


def _layer_body(is_last, agg_ref, h_ref, w_ref, b_ref, g_ref, bt_ref,
                dsrc_ref, ddst_ref, *outs):
    nd = _norm(ddst_ref[0, :, 0] + ddst_ref[1, :, 0])
    a = (agg_ref[0] + agg_ref[1]) * nd[:, None]
    hl = (jnp.dot(a, w_ref[...], preferred_element_type=jnp.float32)
          + b_ref[...])
    hl = jnp.maximum(g_ref[...] * hl + bt_ref[...], 0.0)
    hout = h_ref[...] + hl
    if is_last:
        hg_ref, = outs

        @pl.when(pl.program_id(0) == 0)
        def _():
            hg_ref[...] = jnp.zeros_like(hg_ref)

        hg_ref[...] += jnp.sum(hout, axis=0, keepdims=True) * (1.0 / N)
    else:
        hout_ref, m_ref = outs
        hout_ref[...] = hout
        ns = _norm(dsrc_ref[0, :, 0] + dsrc_ref[1, :, 0])
        m_ref[...] = hout * ns[:, None]


def _make_layer(is_last):
    if is_last:
        out_shape = [jax.ShapeDtypeStruct((1, D), jnp.float32)]
        out_specs = [pl.BlockSpec((1, D), lambda i: (0, 0))]
    else:
        out_shape = [
            jax.ShapeDtypeStruct((N, D), jnp.float32),
            jax.ShapeDtypeStruct((N, D), jnp.float32),
        ]
        out_specs = [
            pl.BlockSpec((R, D), lambda i: (i, 0)),
            pl.BlockSpec((R, D), lambda i: (i, 0)),
        ]
    return pl.pallas_call(
        functools.partial(_layer_body, is_last),
        grid=(G,),
        in_specs=[
            pl.BlockSpec((NCORE, R, D), lambda i: (0, i, 0)),
            pl.BlockSpec((R, D), lambda i: (i, 0)),
            pl.BlockSpec((D, D), lambda i: (0, 0)),
            pl.BlockSpec((1, D), lambda i: (0, 0)),
            pl.BlockSpec((1, D), lambda i: (0, 0)),
            pl.BlockSpec((1, D), lambda i: (0, 0)),
            pl.BlockSpec((NCORE, R, 1), lambda i: (0, i, 0)),
            pl.BlockSpec((NCORE, R, 1), lambda i: (0, i, 0)),
        ],
        out_specs=out_specs,
        out_shape=out_shape,
    )


_layer_mid = _make_layer(False)
_layer_last = _make_layer(True)


def kernel(h, edge_index, e, W_embed, b_embed, Ws, bs, gammas, betas):
    del e
    ei = edge_index.astype(jnp.int32)
    src = ei[0]
    dst = ei[1]
    ones_v = jnp.ones((K,), jnp.float32)
    zeros = jnp.zeros((NP, D), jnp.float32)
    z1 = jnp.zeros((NP,), jnp.float32)
    deg_s, deg_d = _deg_call(src, dst, ones_v, z1)
    deg_s = deg_s.reshape(NCORE, NP, 1)
    deg_d = deg_d.reshape(NCORE, NP, 1)
    hcur, m = _prologue_call(h, W_embed, b_embed.reshape(1, D), deg_s)
    for l in range(L):
        agg = _agg_call(m, src, dst, zeros)
        b2 = bs[l].reshape(1, D)
        g2 = gammas[l].reshape(1, D)
        bt2 = betas[l].reshape(1, D)
        if l < L - 1:
            hcur, m = _layer_mid(agg, hcur, Ws[l], b2, g2, bt2, deg_s,
                                 deg_d)
        else:
            hg, = _layer_last(agg, hcur, Ws[l], b2, g2, bt2, deg_s, deg_d)
    return hg


# speedup vs baseline: 10.4233x; 1.1448x over previous
import functools

import jax
import jax.numpy as jnp
from jax import lax
from jax.experimental import pallas as pl
from jax.experimental.pallas import tpu as pltpu
from jax.experimental.pallas import tpu_sc as plsc

N = 10000
E = 320000
D = 128
H = 64
NP = 10240
L = 4

NCORE = 2
NSUB = 16
EPT = E // NSUB
K = 80
NIT = EPT // K
RPT = NP // NSUB

_MESH = plsc.VectorSubcoreMesh(core_axis_name="c", subcore_axis_name="s")


def _deg_body(src_hbm, dst_hbm, ones_hbm, z_hbm, deg_s_hbm, deg_d_hbm,
              s0, d0, s1, d1, ones_v, deg_s_sh, deg_d_sh,
              is0, id0, is1, id1):
    c = lax.axis_index("c")
    s = lax.axis_index("s")
    pltpu.sync_copy(ones_hbm, ones_v)
    pltpu.sync_copy(z_hbm.at[pl.ds(s * RPT, RPT)],
                    deg_s_sh.at[pl.ds(s * RPT, RPT)])
    pltpu.sync_copy(z_hbm.at[pl.ds(s * RPT, RPT)],
                    deg_d_sh.at[pl.ds(s * RPT, RPT)])
    plsc.subcore_barrier()
    base = c * EPC + s * EPT2

    pltpu.async_copy(src_hbm.at[pl.ds(base, K)], s0, is0)
    pltpu.async_copy(dst_hbm.at[pl.ds(base, K)], d0, id0)

    @pl.loop(0, (NIT2 - 1) // 2)
    def _(j):
        i0 = 2 * j
        pltpu.async_copy(src_hbm.at[pl.ds(base + (i0 + 1) * K, K)], s1, is1)
        pltpu.async_copy(dst_hbm.at[pl.ds(base + (i0 + 1) * K, K)], d1, id1)
        pltpu.make_async_copy(src_hbm.at[pl.ds(base, K)], s0, is0).wait()
        pltpu.make_async_copy(dst_hbm.at[pl.ds(base, K)], d0, id0).wait()
        a = pltpu.async_copy(ones_v, deg_s_sh.at[s0], is0, add=True)
        b = pltpu.async_copy(ones_v, deg_d_sh.at[d0], id0, add=True)
        a.wait()
        b.wait()
        pltpu.async_copy(src_hbm.at[pl.ds(base + (i0 + 2) * K, K)], s0, is0)
        pltpu.async_copy(dst_hbm.at[pl.ds(base + (i0 + 2) * K, K)], d0, id0)
        pltpu.make_async_copy(src_hbm.at[pl.ds(base, K)], s1, is1).wait()
        pltpu.make_async_copy(dst_hbm.at[pl.ds(base, K)], d1, id1).wait()
        a2 = pltpu.async_copy(ones_v, deg_s_sh.at[s1], is1, add=True)
        b2 = pltpu.async_copy(ones_v, deg_d_sh.at[d1], id1, add=True)
        a2.wait()
        b2.wait()

    pltpu.make_async_copy(src_hbm.at[pl.ds(base, K)], s0, is0).wait()
    pltpu.make_async_copy(dst_hbm.at[pl.ds(base, K)], d0, id0).wait()
    a = pltpu.async_copy(ones_v, deg_s_sh.at[s0], is0, add=True)
    b = pltpu.async_copy(ones_v, deg_d_sh.at[d0], id0, add=True)
    a.wait()
    b.wait()

    plsc.subcore_barrier()
    pltpu.sync_copy(deg_s_sh.at[pl.ds(s * RPT, RPT)],
                    deg_s_hbm.at[pl.ds(c * NP + s * RPT, RPT)])
    pltpu.sync_copy(deg_d_sh.at[pl.ds(s * RPT, RPT)],
                    deg_d_hbm.at[pl.ds(c * NP + s * RPT, RPT)])


_deg_call = pl.kernel(
    _deg_body,
    out_type=(jax.ShapeDtypeStruct((NCORE * NP,), jnp.float32),
              jax.ShapeDtypeStruct((NCORE * NP,), jnp.float32)),
    mesh=_MESH,
    scratch_types=[
        pltpu.VMEM((K,), jnp.int32),
        pltpu.VMEM((K,), jnp.int32),
        pltpu.VMEM((K,), jnp.int32),
        pltpu.VMEM((K,), jnp.int32),
        pltpu.VMEM((K,), jnp.float32),
        pltpu.VMEM_SHARED((NP,), jnp.float32),
        pltpu.VMEM_SHARED((NP,), jnp.float32),
        pltpu.SemaphoreType.DMA,
        pltpu.SemaphoreType.DMA,
        pltpu.SemaphoreType.DMA,
        pltpu.SemaphoreType.DMA,
    ],
)


EPC = E // NCORE
EPT2 = EPC // NSUB
NIT2 = EPT2 // K


def _agg_body(m_hbm, src_hbm, dst_hbm, z_hbm, agg_hbm,
              sidx, d0, d1, r0, r1, agg_sh, sg0, sg1, si0, si1):
    c = lax.axis_index("c")
    s = lax.axis_index("s")
    pltpu.sync_copy(z_hbm.at[pl.ds(s * RPT, RPT)],
                    agg_sh.at[pl.ds(s * RPT, RPT)])
    base = c * EPC + s * EPT2
    pltpu.sync_copy(src_hbm.at[pl.ds(base, EPT2)], sidx)
    plsc.subcore_barrier()

    pltpu.async_copy(dst_hbm.at[pl.ds(base, K)], d0, si0)
    pltpu.async_copy(m_hbm.at[sidx.at[pl.ds(0, K)]], r0, sg0)

    @pl.loop(0, (NIT2 - 1) // 2)
    def _(j):
        i0 = 2 * j
        pltpu.async_copy(dst_hbm.at[pl.ds(base + (i0 + 1) * K, K)], d1, si1)
        pltpu.async_copy(m_hbm.at[sidx.at[pl.ds((i0 + 1) * K, K)]], r1, sg1)
        pltpu.make_async_copy(m_hbm.at[pl.ds(0, K)], r0, sg0).wait()
        pltpu.make_async_copy(dst_hbm.at[pl.ds(base, K)], d0, si0).wait()
        pltpu.sync_copy(r0, agg_sh.at[d0], add=True)
        pltpu.async_copy(dst_hbm.at[pl.ds(base + (i0 + 2) * K, K)], d0, si0)
        pltpu.async_copy(m_hbm.at[sidx.at[pl.ds((i0 + 2) * K, K)]], r0, sg0)
        pltpu.make_async_copy(m_hbm.at[pl.ds(0, K)], r1, sg1).wait()
        pltpu.make_async_copy(dst_hbm.at[pl.ds(base, K)], d1, si1).wait()
        pltpu.sync_copy(r1, agg_sh.at[d1], add=True)

    pltpu.make_async_copy(m_hbm.at[pl.ds(0, K)], r0, sg0).wait()
    pltpu.make_async_copy(dst_hbm.at[pl.ds(base, K)], d0, si0).wait()
    pltpu.sync_copy(r0, agg_sh.at[d0], add=True)

    plsc.subcore_barrier()
    pltpu.sync_copy(agg_sh.at[pl.ds(s * RPT, RPT)],
                    agg_hbm.at[c, pl.ds(s * RPT, RPT)])


_agg_call = pl.kernel(
    _agg_body,
    out_type=jax.ShapeDtypeStruct((NCORE, NP, D), jnp.float32),
    mesh=_MESH,
    scratch_types=[
        pltpu.VMEM((EPT2,), jnp.int32),
        pltpu.VMEM((K,), jnp.int32),
        pltpu.VMEM((K,), jnp.int32),
        pltpu.VMEM((K, D), jnp.float32),
        pltpu.VMEM((K, D), jnp.float32),
        pltpu.VMEM_SHARED((NP, D), jnp.float32),
        pltpu.SemaphoreType.DMA,
        pltpu.SemaphoreType.DMA,
        pltpu.SemaphoreType.DMA,
        pltpu.SemaphoreType.DMA,
    ],
)


R = 2000
G = N // R


def _norm(d):
    return jnp.where(d > 0, lax.rsqrt(jnp.maximum(d, 1.0)), 0.0)


def _prologue_body(h_ref, w_ref, b_ref, dsrc_ref, h0_ref, m_ref):
    h0 = jnp.dot(h_ref[...], w_ref[...],
                 preferred_element_type=jnp.float32) + b_ref[...]
    h0_ref[...] = h0
    ns = _norm(dsrc_ref[0, :, 0] + dsrc_ref[1, :, 0])
    m_ref[...] = h0 * ns[:, None]


_prologue_call = pl.pallas_call(
    _prologue_body,
    grid=(G,),
    in_specs=[
        pl.BlockSpec((R, D), lambda i: (i, 0)),
        pl.BlockSpec((D, D), lambda i: (0, 0)),
        pl.BlockSpec((1, D), lambda i: (0, 0)),
        pl.BlockSpec((NCORE, R, 1), lambda i: (0, i, 0)),
    ],
    out_specs=[
        pl.BlockSpec((R, D), lambda i: (i, 0)),
        pl.BlockSpec((R, D), lambda i: (i, 0)),
    ],
    out_shape=[
        jax.ShapeDtypeStruct((N, D), jnp.float32),
        jax.ShapeDtypeStruct((N, D), jnp.float32),
    ],
)


def _layer_body(is_last, agg_ref, h_ref, w_ref, b_ref, g_ref, bt_ref,
                dsrc_ref, ddst_ref, *outs):
    nd = _norm(ddst_ref[0, :, 0] + ddst_ref[1, :, 0])
    a = (agg_ref[0] + agg_ref[1]) * nd[:, None]
    hl = (jnp.dot(a, w_ref[...], preferred_element_type=jnp.float32)
          + b_ref[...])
    hl = jnp.maximum(g_ref[...] * hl + bt_ref[...], 0.0)
    hout = h_ref[...] + hl
    if is_last:
        hg_ref, = outs

        @pl.when(pl.program_id(0) == 0)
        def _():
            hg_ref[...] = jnp.zeros_like(hg_ref)

        hg_ref[...] += jnp.sum(hout, axis=0, keepdims=True) * (1.0 / N)
    else:
        hout_ref, m_ref = outs
        hout_ref[...] = hout
        ns = _norm(dsrc_ref[0, :, 0] + dsrc_ref[1, :, 0])
        m_ref[...] = hout * ns[:, None]


def _make_layer(is_last):
    if is_last:
        out_shape = [jax.ShapeDtypeStruct((1, D), jnp.float32)]
        out_specs = [pl.BlockSpec((1, D), lambda i: (0, 0))]
    else:
        out_shape = [
            jax.ShapeDtypeStruct((N, D), jnp.float32),
            jax.ShapeDtypeStruct((N, D), jnp.float32),
        ]
        out_specs = [
            pl.BlockSpec((R, D), lambda i: (i, 0)),
            pl.BlockSpec((R, D), lambda i: (i, 0)),
        ]
    return pl.pallas_call(
        functools.partial(_layer_body, is_last),
        grid=(G,),
        in_specs=[
            pl.BlockSpec((NCORE, R, D), lambda i: (0, i, 0)),
            pl.BlockSpec((R, D), lambda i: (i, 0)),
            pl.BlockSpec((D, D), lambda i: (0, 0)),
            pl.BlockSpec((1, D), lambda i: (0, 0)),
            pl.BlockSpec((1, D), lambda i: (0, 0)),
            pl.BlockSpec((1, D), lambda i: (0, 0)),
            pl.BlockSpec((NCORE, R, 1), lambda i: (0, i, 0)),
            pl.BlockSpec((NCORE, R, 1), lambda i: (0, i, 0)),
        ],
        out_specs=out_specs,
        out_shape=out_shape,
    )


_layer_mid = _make_layer(False)
_layer_last = _make_layer(True)


def kernel(h, edge_index, e, W_embed, b_embed, Ws, bs, gammas, betas):
    del e
    ei = edge_index.astype(jnp.int32)
    src = ei[0]
    dst = ei[1]
    ones_v = jnp.ones((K,), jnp.float32)
    zeros = jnp.zeros((NP, D), jnp.float32)
    z1 = jnp.zeros((NP,), jnp.float32)
    deg_s, deg_d = _deg_call(src, dst, ones_v, z1)
    deg_s = deg_s.reshape(NCORE, NP, 1)
    deg_d = deg_d.reshape(NCORE, NP, 1)
    hcur, m = _prologue_call(h, W_embed, b_embed.reshape(1, D), deg_s)
    for l in range(L):
        agg = _agg_call(m, src, dst, zeros)
        b2 = bs[l].reshape(1, D)
        g2 = gammas[l].reshape(1, D)
        bt2 = betas[l].reshape(1, D)
        if l < L - 1:
            hcur, m = _layer_mid(agg, hcur, Ws[l], b2, g2, bt2, deg_s,
                                 deg_d)
        else:
            hg, = _layer_last(agg, hcur, Ws[l], b2, g2, bt2, deg_s, deg_d)
    return hg
